# baseline (speedup 1.0000x reference)
import jax
import jax.numpy as jnp
from jax.experimental import pallas as pl
from jax.experimental.pallas import tpu as pltpu

_B, _L, _CLEN, _E, _DM = 16, 64, 512, 8, 1024
_FIN = 3 * _CLEN
_ROWS = _B * _L
_K = _E * _FIN
_NJ = 8
_DT = _DM // _NJ


def _moe_kernel(logits_ref, masks_ref, cc_hbm, w_ref, b_ref, out_ref, gl_ref,
                flat32_ref, xg_ref, sems):
    j = pl.program_id(0)

    logits = logits_ref[...]
    mask = (masks_ref[...] == 1).astype(jnp.float32)
    m = jnp.max(logits, axis=1, keepdims=True)
    ex = jnp.exp(logits - m)
    sm = ex / jnp.sum(ex, axis=1, keepdims=True)
    g = sm * mask
    g = g / (jnp.sum(g, axis=1, keepdims=True) + 1e-9)

    row_b = jax.lax.broadcasted_iota(jnp.int32, (_ROWS, _B), 0) // _L
    col_b = jax.lax.broadcasted_iota(jnp.int32, (_ROWS, _B), 1)
    P = (row_b == col_b).astype(jnp.float32)
    geall = jnp.dot(P, g, preferred_element_type=jnp.float32)

    @pl.when(j == 0)
    def _():
        s = jnp.sum(sm * mask) / _B
        gl_ref[0, 0] = (1.0 - s) * (1.0 - s)
        for c in range(3):
            pltpu.make_async_copy(
                cc_hbm.at[:, :, c, 0, :],
                flat32_ref.at[:, :, c * _CLEN:(c + 1) * _CLEN],
                sems.at[c],
            ).start()
        for c in range(3):
            pltpu.make_async_copy(
                cc_hbm.at[:, :, c, 0, :],
                flat32_ref.at[:, :, c * _CLEN:(c + 1) * _CLEN],
                sems.at[c],
            ).wait()
        flat2 = flat32_ref[...].reshape(_ROWS, _FIN)
        for e in range(_E):
            xg_ref[:, e * _FIN:(e + 1) * _FIN] = (
                geall[:, e:e + 1] * flat2).astype(jnp.bfloat16)

    wb = w_ref[...].astype(jnp.bfloat16)
    big = jnp.dot(xg_ref[...], wb, preferred_element_type=jnp.float32)
    bias = jnp.dot(geall, b_ref[...], preferred_element_type=jnp.float32)
    out_ref[...] = (big + bias).reshape(_B, _L, _DT).astype(jnp.bfloat16)


def kernel(cycle_curve_data, logits, moe_masks, expert_w, expert_b):
    out, gl = pl.pallas_call(
        _moe_kernel,
        grid=(_NJ,),
        in_specs=[
            pl.BlockSpec((_B, _E), lambda j: (0, 0)),
            pl.BlockSpec((_B, _E), lambda j: (0, 0)),
            pl.BlockSpec(memory_space=pl.MemorySpace.ANY),
            pl.BlockSpec((_K, _DT), lambda j: (0, j)),
            pl.BlockSpec((_E, _DT), lambda j: (0, j)),
        ],
        out_specs=[
            pl.BlockSpec((_B, _L, _DT), lambda j: (0, 0, j)),
            pl.BlockSpec(memory_space=pltpu.SMEM),
        ],
        out_shape=[
            jax.ShapeDtypeStruct((_B, _L, _DM), jnp.bfloat16),
            jax.ShapeDtypeStruct((1, 1), jnp.float32),
        ],
        scratch_shapes=[
            pltpu.VMEM((_B, _L, _FIN), jnp.float32),
            pltpu.VMEM((_ROWS, _K), jnp.bfloat16),
            pltpu.SemaphoreType.DMA((3,)),
        ],
    )(logits, moe_masks, cycle_curve_data.reshape(_B, _L, 3, 1, _CLEN),
      expert_w.reshape(_K, _DM), expert_b)

    return out, gl.reshape(())
